# Initial kernel scaffold; baseline (speedup 1.0000x reference)
import jax
import jax.numpy as jnp
from jax.experimental import pallas as pl


def kernel(edges, attentions, N, X):
    raise NotImplementedError("write your pallas kernel here")


# speedup vs baseline: 4.0205x; 4.0205x over previous
import functools

import jax
import jax.numpy as jnp
from jax import lax
from jax.experimental import pallas as pl
from jax.experimental.pallas import tpu as pltpu
from jax.experimental.pallas import tpu_sc as plsc

_NC = 2
_NS = 16
_CHUNK = 128


@functools.partial(jax.jit, static_argnums=(3, 4))
def _spmm_sc(edges_p, attn_p, x, n_nodes, dim):
    e_pad = attn_p.shape[0]
    epw = e_pad // (_NC * _NS)
    n_chunks = epw // _CHUNK
    rows_per_sub = (n_nodes // _NS) // 8 * 8
    tail_rows = n_nodes - _NS * rows_per_sub
    tail_r0 = _NS * rows_per_sub

    mesh = plsc.VectorSubcoreMesh(core_axis_name="c", subcore_axis_name="s")

    @functools.partial(
        pl.kernel,
        out_type=jax.ShapeDtypeStruct((_NC, n_nodes, dim), jnp.float32),
        mesh=mesh,
        scratch_types=[
            pltpu.VMEM((_CHUNK,), jnp.int32),
            pltpu.VMEM((_CHUNK,), jnp.int32),
            pltpu.VMEM((_CHUNK,), jnp.float32),
            pltpu.VMEM((_CHUNK, dim), jnp.float32),
            pltpu.VMEM_SHARED((n_nodes, dim), jnp.float32),
            pltpu.SemaphoreType.DMA,
        ],
    )
    def k(edges_hbm, attn_hbm, x_hbm, zeros_hbm, out_hbm,
          src_v, dst_v, attn_v, rows_v, acc_sh, sem):
        c = lax.axis_index("c")
        s = lax.axis_index("s")
        wid = c * _NS + s

        r0 = s * rows_per_sub
        pltpu.sync_copy(zeros_hbm.at[pl.ds(r0, rows_per_sub)],
                        acc_sh.at[pl.ds(r0, rows_per_sub)])
        if tail_rows:
            @pl.when(s == _NS - 1)
            def _():
                pltpu.sync_copy(zeros_hbm.at[pl.ds(tail_r0, tail_rows)],
                                acc_sh.at[pl.ds(tail_r0, tail_rows)])
        plsc.subcore_barrier()

        base_w = wid * epw

        def chunk_body(i, carry):
            base = base_w + i * _CHUNK
            pltpu.sync_copy(edges_hbm.at[0, pl.ds(base, _CHUNK)], src_v)
            pltpu.sync_copy(edges_hbm.at[1, pl.ds(base, _CHUNK)], dst_v)
            pltpu.sync_copy(attn_hbm.at[pl.ds(base, _CHUNK)], attn_v)
            pltpu.async_copy(x_hbm.at[dst_v], rows_v, sem).wait()

            dnums = lax.GatherDimensionNumbers(
                offset_dims=(), collapsed_slice_dims=(0,), start_index_map=(0,))

            def scale_group(g, carry2):
                av = attn_v[pl.ds(g * 16, 16)]
                for r in range(16):
                    row = g * 16 + r
                    lane_idx = (jnp.zeros((16,), jnp.int32) + r)[:, None]
                    a = lax.gather(av, lane_idx, dnums, (1,),
                                   mode=lax.GatherScatterMode.PROMISE_IN_BOUNDS)
                    for cb in range(dim // 16):
                        sl = pl.ds(cb * 16, 16)
                        rows_v[row, sl] = rows_v[row, sl] * a
                return carry2

            lax.fori_loop(0, _CHUNK // 16, scale_group, 0)
            pltpu.sync_copy(rows_v, acc_sh.at[src_v], add=True)
            return carry

        lax.fori_loop(0, n_chunks, chunk_body, 0)

        plsc.subcore_barrier()
        pltpu.sync_copy(acc_sh.at[pl.ds(r0, rows_per_sub)],
                        out_hbm.at[c, pl.ds(r0, rows_per_sub)])
        if tail_rows:
            @pl.when(s == _NS - 1)
            def _():
                pltpu.sync_copy(acc_sh.at[pl.ds(tail_r0, tail_rows)],
                                out_hbm.at[c, pl.ds(tail_r0, tail_rows)])

    zeros = jnp.zeros((n_nodes, dim), jnp.float32)
    return k(edges_p, attn_p, x, zeros)


def _combine_tc(parts, bias):
    nc, n_nodes, dim = parts.shape
    blk = 1000

    def body(bias_ref, p_ref, o_ref):
        o_ref[...] = p_ref[0] + p_ref[1] + bias_ref[0]

    return pl.pallas_call(
        body,
        grid=(n_nodes // blk,),
        in_specs=[
            pl.BlockSpec(memory_space=pltpu.SMEM),
            pl.BlockSpec((nc, blk, dim), lambda i: (0, i, 0)),
        ],
        out_specs=pl.BlockSpec((blk, dim), lambda i: (i, 0)),
        out_shape=jax.ShapeDtypeStruct((n_nodes, dim), jnp.float32),
    )(bias, parts)


def kernel(edges, attentions, N, X):
    n_nodes, dim = X.shape
    e = attentions.shape[0]
    per_worker = -(-e // (_NC * _NS * _CHUNK)) * _CHUNK
    pad = per_worker * _NC * _NS - e
    edges_p = jnp.concatenate(
        [edges.astype(jnp.int32), jnp.zeros((2, pad), jnp.int32)], axis=1)
    attn_p = jnp.concatenate(
        [attentions.astype(jnp.float32), jnp.zeros((pad,), jnp.float32)])
    parts = _spmm_sc(edges_p, attn_p, X, n_nodes, dim)
    bias = (jnp.asarray(N, jnp.float32) - jnp.float32(n_nodes)).reshape(1)
    return _combine_tc(parts, bias)
